# baseline (speedup 1.0000x reference)
import dataclasses

import jax
import jax.numpy as jnp
from jax import lax
from jax.experimental import pallas as pl
from jax.experimental.pallas import tpu as pltpu
from jax.experimental.pallas import tpu_sc as plsc

D = 128
CZ, CY, CX = 4, 26, 26
CZYX = CZ * CY * CX
CYX = CY * CX
NCANVAS = CZYX * 4
MP = 50000
M = 200000

_MESH = plsc.VectorSubcoreMesh(core_axis_name="c", subcore_axis_name="s")

_CP = pltpu.CompilerParams()
if "needs_layout_passes" in pltpu.CompilerParams.__dataclass_fields__:
    _CP = dataclasses.replace(_CP, needs_layout_passes=False)

A1_CNT = 12512
A2_CNT = 3136
SLOT_CNT = 680
A4_CNT = 6256
A4_SIZES = (1568, 1568, 1568, 1552)
NQ = 4
MQ = M // NQ
B_CNT = 1568
B_CHUNK = 128
G_STAGE = 680
BM = 5000
BLOCKS_Q = MQ // BM


def _idx_body(w0, w1, w2, w3, c0, c1, c2, c3, f2b, gix_out,
              win0buf, c0b, c1b, c2b, c3b, canvas, vtmp, m16buf, mergebuf,
              f2bbuf, tpart, gixbuf, shmax, shcanvas, tsh, semf):
    c = lax.axis_index("c")
    s = lax.axis_index("s")
    iota = lax.iota(jnp.int32, 16)

    base1 = jnp.where(s == 15, M - A1_CNT, s * A1_CNT)
    pltpu.sync_copy(w0.at[pl.ds(base1, A1_CNT)], win0buf)

    def a1_step(o, m):
        return jnp.maximum(m, win0buf[pl.ds(o * 16, 16)])
    m = lax.fori_loop(0, A1_CNT // 16, a1_step, jnp.zeros((16,), jnp.int32))
    vtmp[...] = m
    pltpu.sync_copy(vtmp, shmax.at[pl.ds(s * 16, 16)])
    plsc.subcore_barrier()
    pltpu.sync_copy(shmax, m16buf)
    mm = m16buf[pl.ds(0, 16)]
    for r in range(1, 16):
        mm = jnp.maximum(mm, m16buf[pl.ds(r * 16, 16)])
    canvas_len = CZYX * (jnp.max(mm) + 1)

    @pl.loop(0, NCANVAS // 16)
    def _(o):
        canvas[pl.ds(o * 16, 16)] = jnp.zeros((16,), jnp.int32)

    base2 = jnp.where(s == 15, MP - A2_CNT, s * A2_CNT)
    pltpu.sync_copy(c0.at[pl.ds(base2, A2_CNT)], c0b)
    pltpu.sync_copy(c1.at[pl.ds(base2, A2_CNT)], c1b)
    pltpu.sync_copy(c2.at[pl.ds(base2, A2_CNT)], c2b)
    pltpu.sync_copy(c3.at[pl.ds(base2, A2_CNT)], c3b)

    lane_masks = [iota == j for j in range(16)]

    @pl.loop(0, A2_CNT // 16)
    def _(o):
        sl = pl.ds(o * 16, 16)
        idx = (c0b[sl] * CZYX + c1b[sl] * CYX + c2b[sl] * CX + c3b[sl])
        vals = (base2 + o * 16 + 1) + iota
        valid = idx < canvas_len
        for j in range(16):
            plsc.store_scatter(canvas, [idx], vals,
                               mask=jnp.logical_and(valid, lane_masks[j]))

    pltpu.sync_copy(canvas, shcanvas.at[pl.ds(s * NCANVAS, NCANVAS)])
    plsc.subcore_barrier()

    sbase = jnp.where(s == 15, NCANVAS - SLOT_CNT, s * SLOT_CNT)
    fcp = pltpu.async_copy(f2b, f2bbuf, semf)
    for r in range(16):
        pltpu.sync_copy(shcanvas.at[pl.ds(r * NCANVAS + sbase, SLOT_CNT)],
                        mergebuf.at[pl.ds(r * SLOT_CNT, SLOT_CNT)])
    fcp.wait()

    @pl.loop(0, SLOT_CNT // 16 + 1)
    def _(o):
        off = jnp.where(o == SLOT_CNT // 16, SLOT_CNT - 16, o * 16)
        mv = mergebuf[pl.ds(off, 16)]
        for r in range(1, 16):
            mv = jnp.maximum(mv, mergebuf[pl.ds(r * SLOT_CNT + off, 16)])
        row = jnp.maximum(mv - 1, 0)
        tpart[pl.ds(off, 16)] = plsc.load_gather(f2bbuf, [row])

    pltpu.sync_copy(tpart.at[pl.ds(0, SLOT_CNT)], tsh.at[pl.ds(sbase, SLOT_CNT)])
    plsc.subcore_barrier()

    pltpu.sync_copy(tsh, canvas)
    wid = c * 16 + s
    tbase = jnp.where(wid == 31, M - A4_CNT, wid * A4_CNT)
    coff = 0
    for sz in A4_SIZES:
        pltpu.sync_copy(w0.at[pl.ds(tbase + coff, sz)], c0b.at[pl.ds(0, sz)])
        pltpu.sync_copy(w1.at[pl.ds(tbase + coff, sz)], c1b.at[pl.ds(0, sz)])
        pltpu.sync_copy(w2.at[pl.ds(tbase + coff, sz)], c2b.at[pl.ds(0, sz)])
        pltpu.sync_copy(w3.at[pl.ds(tbase + coff, sz)], c3b.at[pl.ds(0, sz)])

        @pl.loop(0, sz // 16)
        def _(o):
            sl = pl.ds(o * 16, 16)
            vi = (c0b[sl] * CZYX + c1b[sl] * CYX + c2b[sl] * CX + c3b[sl])
            gixbuf[sl] = plsc.load_gather(canvas, [vi])

        pltpu.sync_copy(gixbuf.at[pl.ds(0, sz)],
                        gix_out.at[pl.ds(tbase + coff, sz)])
        coff += sz


def _make_prop_body(q):
    def _prop_body(gix, g_hbm, p_out,
                   gixb, rows0, rows1, gshared,
                   semstage, semg0, semg1, semo0, semo1):
        c = lax.axis_index("c")
        s = lax.axis_index("s")
        wid = c * 16 + s
        tbase = jnp.where(wid == 31, MQ - B_CNT, wid * B_CNT)
        gbase = q * MQ + tbase

        grow = jnp.where(s == 15, NCANVAS - G_STAGE, s * G_STAGE)
        stg = pltpu.async_copy(g_hbm.at[pl.ds(grow, G_STAGE), :],
                               gshared.at[pl.ds(grow, G_STAGE), :], semstage)
        pltpu.sync_copy(gix.at[pl.ds(gbase, B_CNT)], gixb)
        stg.wait()
        plsc.subcore_barrier()

        offs = [k * B_CHUNK for k in range(B_CNT // B_CHUNK)]
        if offs[-1] + B_CHUNK < B_CNT:
            offs.append(B_CNT - B_CHUNK)
        rows = [rows0, rows1]
        semg = [semg0, semg1]
        semo = [semo0, semo1]

        def fire_gather(off, par):
            return pltpu.async_copy(
                gshared.at[gixb.at[pl.ds(off, B_CHUNK)]],
                rows[par], semg[par])

        gcp = {0: fire_gather(offs[0], 0)}
        ocp = {}
        n = len(offs)
        for i in range(n):
            par = i % 2
            if i + 1 < n:
                if i >= 1:
                    ocp.pop(i - 1).wait()
                gcp[i + 1] = fire_gather(offs[i + 1], 1 - par)
            gcp.pop(i).wait()
            ocp[i] = pltpu.async_copy(
                rows[par], p_out.at[pl.ds(tbase + offs[i], B_CHUNK), :],
                semo[par])
        ocp.pop(n - 2).wait()
        ocp.pop(n - 1).wait()
    return _prop_body


def _fuse_body(vf_ref, p_ref, w1t_ref, gamma_ref, beta_ref, o_ref):
    y = jnp.dot(vf_ref[...].astype(jnp.bfloat16),
                w1t_ref[...].astype(jnp.bfloat16),
                preferred_element_type=jnp.float32) + p_ref[...]
    ones = jnp.ones((D, 1), jnp.float32)
    s1 = jnp.dot(y, ones, preferred_element_type=jnp.float32)
    s2 = jnp.dot(y * y, ones, preferred_element_type=jnp.float32)
    mean = s1 * (1.0 / D)
    var = s2 * (1.0 / D) - mean * mean
    rstd = jax.lax.rsqrt(var + 1e-5)
    scale = rstd * gamma_ref[...]
    shift = beta_ref[...] - mean * scale
    o_ref[...] = jnp.maximum(y * scale + shift, 0.0)


def _fuse_body_alias(vf_ref, p_ref, w1t_ref, gamma_ref, beta_ref, prev_ref,
                     o_ref):
    del prev_ref
    _fuse_body(vf_ref, p_ref, w1t_ref, gamma_ref, beta_ref, o_ref)


def _g_body(flat_ref, w2t_ref, g_ref):
    g_ref[...] = jnp.dot(flat_ref[...].astype(jnp.bfloat16),
                         w2t_ref[...].astype(jnp.bfloat16),
                         preferred_element_type=jnp.float32)


def kernel(voxel_feat_win_batch, flat2batch_inds, voxel_coord_win, voxel_feat,
           win_inds_bzyx_interreg, W, gamma, beta):
    flat = voxel_feat_win_batch.reshape(-1, D)
    NF = flat.shape[0]

    Wt = W.T
    W1T = Wt[:D, :]
    W2T = Wt[D:, :]
    gamma2 = gamma.reshape(1, D)
    beta2 = beta.reshape(1, D)

    w_cols = [jnp.asarray(win_inds_bzyx_interreg[:, j]) for j in range(4)]
    c_cols = [jnp.asarray(voxel_coord_win[:, j]) for j in range(4)]

    idx_kernel = pl.kernel(
        _idx_body,
        out_type=jax.ShapeDtypeStruct((M,), jnp.int32),
        mesh=_MESH,
        scratch_types=[
            pltpu.VMEM((A1_CNT,), jnp.int32),
            pltpu.VMEM((A2_CNT,), jnp.int32),
            pltpu.VMEM((A2_CNT,), jnp.int32),
            pltpu.VMEM((A2_CNT,), jnp.int32),
            pltpu.VMEM((A2_CNT,), jnp.int32),
            pltpu.VMEM((NCANVAS,), jnp.int32),
            pltpu.VMEM((16,), jnp.int32),
            pltpu.VMEM((256,), jnp.int32),
            pltpu.VMEM((16 * SLOT_CNT,), jnp.int32),
            pltpu.VMEM((MP,), jnp.int32),
            pltpu.VMEM((768,), jnp.int32),
            pltpu.VMEM((A4_SIZES[0],), jnp.int32),
            pltpu.VMEM_SHARED((256,), jnp.int32),
            pltpu.VMEM_SHARED((16 * NCANVAS,), jnp.int32),
            pltpu.VMEM_SHARED((NCANVAS,), jnp.int32),
            pltpu.SemaphoreType.DMA,
        ],
        compiler_params=_CP,
    )
    gix = idx_kernel(w_cols[0], w_cols[1], w_cols[2], w_cols[3],
                     c_cols[0], c_cols[1], c_cols[2], c_cols[3],
                     flat2batch_inds)

    G = pl.pallas_call(
        _g_body,
        out_shape=jax.ShapeDtypeStruct((NF, D), jnp.float32),
    )(flat, W2T)

    sc_scratch = [
        pltpu.VMEM((B_CNT,), jnp.int32),
        pltpu.VMEM((B_CHUNK, D), jnp.float32),
        pltpu.VMEM((B_CHUNK, D), jnp.float32),
        pltpu.VMEM_SHARED((NCANVAS, D), jnp.float32),
        pltpu.SemaphoreType.DMA,
        pltpu.SemaphoreType.DMA,
        pltpu.SemaphoreType.DMA,
        pltpu.SemaphoreType.DMA,
        pltpu.SemaphoreType.DMA,
    ]
    out = None
    for q in range(NQ):
        prop_kernel = pl.kernel(
            _make_prop_body(q),
            out_type=jax.ShapeDtypeStruct((MQ, D), jnp.float32),
            mesh=_MESH,
            scratch_types=sc_scratch,
            compiler_params=_CP,
        )
        p_q = prop_kernel(gix, G)

        vf_spec = pl.BlockSpec((BM, D), lambda i, q=q: (q * BLOCKS_Q + i, 0))
        common_specs = [
            vf_spec,
            pl.BlockSpec((BM, D), lambda i: (i, 0)),
            pl.BlockSpec((D, D), lambda i: (0, 0)),
            pl.BlockSpec((1, D), lambda i: (0, 0)),
            pl.BlockSpec((1, D), lambda i: (0, 0)),
        ]
        out_spec = pl.BlockSpec((BM, D), lambda i, q=q: (q * BLOCKS_Q + i, 0))
        if q == 0:
            out = pl.pallas_call(
                _fuse_body,
                grid=(BLOCKS_Q,),
                in_specs=common_specs,
                out_specs=out_spec,
                out_shape=jax.ShapeDtypeStruct((M, D), jnp.float32),
            )(voxel_feat, p_q, W1T, gamma2, beta2)
        else:
            out = pl.pallas_call(
                _fuse_body_alias,
                grid=(BLOCKS_Q,),
                in_specs=common_specs + [
                    pl.BlockSpec(memory_space=pltpu.MemorySpace.HBM)],
                out_specs=out_spec,
                out_shape=jax.ShapeDtypeStruct((M, D), jnp.float32),
                input_output_aliases={5: 0},
            )(voxel_feat, p_q, W1T, gamma2, beta2, out)
    return out


# speedup vs baseline: 1.0646x; 1.0646x over previous
import dataclasses

import jax
import jax.numpy as jnp
from jax import lax
from jax.experimental import pallas as pl
from jax.experimental.pallas import tpu as pltpu
from jax.experimental.pallas import tpu_sc as plsc

D = 128
CZ, CY, CX = 4, 26, 26
CZYX = CZ * CY * CX
CYX = CY * CX
NCANVAS = CZYX * 4
MP = 50000
M = 200000

_MESH = plsc.VectorSubcoreMesh(core_axis_name="c", subcore_axis_name="s")

_CP = pltpu.CompilerParams()
if "needs_layout_passes" in pltpu.CompilerParams.__dataclass_fields__:
    _CP = dataclasses.replace(_CP, needs_layout_passes=False)

A1_CNT = 12512
A2_CNT = 3136
SLOT_CNT = 680
A4_CNT = 6256
A4_SIZES = (1568, 1568, 1568, 1552)
NQ = 4
MQ = M // NQ
B_CNT = 1568
B_CHUNK = 128
G_STAGE = 680
BM = 5000
BLOCKS_Q = MQ // BM


def _idx_body(w0, w1, w2, w3, c0, c1, c2, c3, f2b, gix_out,
              win0buf, c0b, c1b, c2b, c3b, canvas, vtmp, m16buf, mergebuf,
              f2bbuf, tpart, gixbuf, shmax, shcanvas, tsh, semf):
    c = lax.axis_index("c")
    s = lax.axis_index("s")
    iota = lax.iota(jnp.int32, 16)

    base1 = jnp.where(s == 15, M - A1_CNT, s * A1_CNT)
    pltpu.sync_copy(w0.at[pl.ds(base1, A1_CNT)], win0buf)

    def a1_step(o, m):
        return jnp.maximum(m, win0buf[pl.ds(o * 16, 16)])
    m = lax.fori_loop(0, A1_CNT // 16, a1_step, jnp.zeros((16,), jnp.int32))
    vtmp[...] = m
    pltpu.sync_copy(vtmp, shmax.at[pl.ds(s * 16, 16)])
    plsc.subcore_barrier()
    pltpu.sync_copy(shmax, m16buf)
    mm = m16buf[pl.ds(0, 16)]
    for r in range(1, 16):
        mm = jnp.maximum(mm, m16buf[pl.ds(r * 16, 16)])
    canvas_len = CZYX * (jnp.max(mm) + 1)

    @pl.loop(0, NCANVAS // 16)
    def _(o):
        canvas[pl.ds(o * 16, 16)] = jnp.zeros((16,), jnp.int32)

    base2 = jnp.where(s == 15, MP - A2_CNT, s * A2_CNT)
    pltpu.sync_copy(c0.at[pl.ds(base2, A2_CNT)], c0b)
    pltpu.sync_copy(c1.at[pl.ds(base2, A2_CNT)], c1b)
    pltpu.sync_copy(c2.at[pl.ds(base2, A2_CNT)], c2b)
    pltpu.sync_copy(c3.at[pl.ds(base2, A2_CNT)], c3b)

    lane_masks = [iota == j for j in range(16)]

    @pl.loop(0, A2_CNT // 16)
    def _(o):
        sl = pl.ds(o * 16, 16)
        idx = (c0b[sl] * CZYX + c1b[sl] * CYX + c2b[sl] * CX + c3b[sl])
        vals = (base2 + o * 16 + 1) + iota
        valid = idx < canvas_len
        for j in range(16):
            plsc.store_scatter(canvas, [idx], vals,
                               mask=jnp.logical_and(valid, lane_masks[j]))

    pltpu.sync_copy(canvas, shcanvas.at[pl.ds(s * NCANVAS, NCANVAS)])
    plsc.subcore_barrier()

    sbase = jnp.where(s == 15, NCANVAS - SLOT_CNT, s * SLOT_CNT)
    fcp = pltpu.async_copy(f2b, f2bbuf, semf)
    for r in range(16):
        pltpu.sync_copy(shcanvas.at[pl.ds(r * NCANVAS + sbase, SLOT_CNT)],
                        mergebuf.at[pl.ds(r * SLOT_CNT, SLOT_CNT)])
    fcp.wait()

    @pl.loop(0, SLOT_CNT // 16 + 1)
    def _(o):
        off = jnp.where(o == SLOT_CNT // 16, SLOT_CNT - 16, o * 16)
        mv = mergebuf[pl.ds(off, 16)]
        for r in range(1, 16):
            mv = jnp.maximum(mv, mergebuf[pl.ds(r * SLOT_CNT + off, 16)])
        row = jnp.maximum(mv - 1, 0)
        tpart[pl.ds(off, 16)] = plsc.load_gather(f2bbuf, [row])

    pltpu.sync_copy(tpart.at[pl.ds(0, SLOT_CNT)], tsh.at[pl.ds(sbase, SLOT_CNT)])
    plsc.subcore_barrier()

    pltpu.sync_copy(tsh, canvas)
    wid = c * 16 + s
    tbase = jnp.where(wid == 31, M - A4_CNT, wid * A4_CNT)
    coff = 0
    for sz in A4_SIZES:
        pltpu.sync_copy(w0.at[pl.ds(tbase + coff, sz)], c0b.at[pl.ds(0, sz)])
        pltpu.sync_copy(w1.at[pl.ds(tbase + coff, sz)], c1b.at[pl.ds(0, sz)])
        pltpu.sync_copy(w2.at[pl.ds(tbase + coff, sz)], c2b.at[pl.ds(0, sz)])
        pltpu.sync_copy(w3.at[pl.ds(tbase + coff, sz)], c3b.at[pl.ds(0, sz)])

        @pl.loop(0, sz // 16)
        def _(o):
            sl = pl.ds(o * 16, 16)
            vi = (c0b[sl] * CZYX + c1b[sl] * CYX + c2b[sl] * CX + c3b[sl])
            gixbuf[sl] = plsc.load_gather(canvas, [vi])

        pltpu.sync_copy(gixbuf.at[pl.ds(0, sz)],
                        gix_out.at[pl.ds(tbase + coff, sz)])
        coff += sz


def _make_prop_body(q):
    def _prop_body(gix, g_hbm, p_out,
                   gixb, rows0, rows1, gshared,
                   semstage, semg0, semg1, semo0, semo1):
        c = lax.axis_index("c")
        s = lax.axis_index("s")
        wid = c * 16 + s
        tbase = jnp.where(wid == 31, MQ - B_CNT, wid * B_CNT)
        gbase = q * MQ + tbase

        grow = jnp.where(s == 15, NCANVAS - G_STAGE, s * G_STAGE)
        stg = pltpu.async_copy(g_hbm.at[pl.ds(grow, G_STAGE), :],
                               gshared.at[pl.ds(grow, G_STAGE), :], semstage)
        pltpu.sync_copy(gix.at[pl.ds(gbase, B_CNT)], gixb)
        stg.wait()
        plsc.subcore_barrier()

        offs = [k * B_CHUNK for k in range(B_CNT // B_CHUNK)]
        if offs[-1] + B_CHUNK < B_CNT:
            offs.append(B_CNT - B_CHUNK)
        rows = [rows0, rows1]
        semg = [semg0, semg1]
        semo = [semo0, semo1]

        def fire_gather(off, par):
            return pltpu.async_copy(
                gshared.at[gixb.at[pl.ds(off, B_CHUNK)]],
                rows[par], semg[par])

        gcp = {0: fire_gather(offs[0], 0)}
        ocp = {}
        n = len(offs)
        for i in range(n):
            par = i % 2
            if i + 1 < n:
                if i >= 1:
                    ocp.pop(i - 1).wait()
                gcp[i + 1] = fire_gather(offs[i + 1], 1 - par)
            gcp.pop(i).wait()
            ocp[i] = pltpu.async_copy(
                rows[par], p_out.at[pl.ds(tbase + offs[i], B_CHUNK), :],
                semo[par])
        ocp.pop(n - 2).wait()
        ocp.pop(n - 1).wait()
    return _prop_body


def _fuse_body(vf_ref, p_ref, w1t_ref, gamma_ref, beta_ref, o_ref):
    y = jnp.dot(vf_ref[...].astype(jnp.bfloat16),
                w1t_ref[...].astype(jnp.bfloat16),
                preferred_element_type=jnp.float32) + p_ref[...]
    s1 = jnp.sum(y, axis=1, keepdims=True)
    s2 = jnp.sum(y * y, axis=1, keepdims=True)
    mean = s1 * (1.0 / D)
    var = s2 * (1.0 / D) - mean * mean
    rstd = jax.lax.rsqrt(var + 1e-5)
    scale = rstd * gamma_ref[...]
    shift = beta_ref[...] - mean * scale
    o_ref[...] = jnp.maximum(y * scale + shift, 0.0)


def _fuse_body_alias(vf_ref, p_ref, w1t_ref, gamma_ref, beta_ref, prev_ref,
                     o_ref):
    del prev_ref
    _fuse_body(vf_ref, p_ref, w1t_ref, gamma_ref, beta_ref, o_ref)


def _g_body(flat_ref, w2t_ref, g_ref):
    g_ref[...] = jnp.dot(flat_ref[...].astype(jnp.bfloat16),
                         w2t_ref[...].astype(jnp.bfloat16),
                         preferred_element_type=jnp.float32)


def kernel(voxel_feat_win_batch, flat2batch_inds, voxel_coord_win, voxel_feat,
           win_inds_bzyx_interreg, W, gamma, beta):
    flat = voxel_feat_win_batch.reshape(-1, D)
    NF = flat.shape[0]

    Wt = W.T
    W1T = Wt[:D, :]
    W2T = Wt[D:, :]
    gamma2 = gamma.reshape(1, D)
    beta2 = beta.reshape(1, D)

    w_cols = [jnp.asarray(win_inds_bzyx_interreg[:, j]) for j in range(4)]
    c_cols = [jnp.asarray(voxel_coord_win[:, j]) for j in range(4)]

    idx_kernel = pl.kernel(
        _idx_body,
        out_type=jax.ShapeDtypeStruct((M,), jnp.int32),
        mesh=_MESH,
        scratch_types=[
            pltpu.VMEM((A1_CNT,), jnp.int32),
            pltpu.VMEM((A2_CNT,), jnp.int32),
            pltpu.VMEM((A2_CNT,), jnp.int32),
            pltpu.VMEM((A2_CNT,), jnp.int32),
            pltpu.VMEM((A2_CNT,), jnp.int32),
            pltpu.VMEM((NCANVAS,), jnp.int32),
            pltpu.VMEM((16,), jnp.int32),
            pltpu.VMEM((256,), jnp.int32),
            pltpu.VMEM((16 * SLOT_CNT,), jnp.int32),
            pltpu.VMEM((MP,), jnp.int32),
            pltpu.VMEM((768,), jnp.int32),
            pltpu.VMEM((A4_SIZES[0],), jnp.int32),
            pltpu.VMEM_SHARED((256,), jnp.int32),
            pltpu.VMEM_SHARED((16 * NCANVAS,), jnp.int32),
            pltpu.VMEM_SHARED((NCANVAS,), jnp.int32),
            pltpu.SemaphoreType.DMA,
        ],
        compiler_params=_CP,
    )
    gix = idx_kernel(w_cols[0], w_cols[1], w_cols[2], w_cols[3],
                     c_cols[0], c_cols[1], c_cols[2], c_cols[3],
                     flat2batch_inds)

    G = pl.pallas_call(
        _g_body,
        out_shape=jax.ShapeDtypeStruct((NF, D), jnp.float32),
    )(flat, W2T)

    sc_scratch = [
        pltpu.VMEM((B_CNT,), jnp.int32),
        pltpu.VMEM((B_CHUNK, D), jnp.float32),
        pltpu.VMEM((B_CHUNK, D), jnp.float32),
        pltpu.VMEM_SHARED((NCANVAS, D), jnp.float32),
        pltpu.SemaphoreType.DMA,
        pltpu.SemaphoreType.DMA,
        pltpu.SemaphoreType.DMA,
        pltpu.SemaphoreType.DMA,
        pltpu.SemaphoreType.DMA,
    ]
    out = None
    for q in range(NQ):
        prop_kernel = pl.kernel(
            _make_prop_body(q),
            out_type=jax.ShapeDtypeStruct((MQ, D), jnp.float32),
            mesh=_MESH,
            scratch_types=sc_scratch,
            compiler_params=_CP,
        )
        p_q = prop_kernel(gix, G)

        vf_spec = pl.BlockSpec((BM, D), lambda i, q=q: (q * BLOCKS_Q + i, 0))
        common_specs = [
            vf_spec,
            pl.BlockSpec((BM, D), lambda i: (i, 0)),
            pl.BlockSpec((D, D), lambda i: (0, 0)),
            pl.BlockSpec((1, D), lambda i: (0, 0)),
            pl.BlockSpec((1, D), lambda i: (0, 0)),
        ]
        out_spec = pl.BlockSpec((BM, D), lambda i, q=q: (q * BLOCKS_Q + i, 0))
        if q == 0:
            out = pl.pallas_call(
                _fuse_body,
                grid=(BLOCKS_Q,),
                in_specs=common_specs,
                out_specs=out_spec,
                out_shape=jax.ShapeDtypeStruct((M, D), jnp.float32),
            )(voxel_feat, p_q, W1T, gamma2, beta2)
        else:
            out = pl.pallas_call(
                _fuse_body_alias,
                grid=(BLOCKS_Q,),
                in_specs=common_specs + [
                    pl.BlockSpec(memory_space=pltpu.MemorySpace.HBM)],
                out_specs=out_spec,
                out_shape=jax.ShapeDtypeStruct((M, D), jnp.float32),
                input_output_aliases={5: 0},
            )(voxel_feat, p_q, W1T, gamma2, beta2, out)
    return out
